# baseline (speedup 1.0000x reference)
import functools

import jax
import jax.numpy as jnp
from jax import lax
from jax.experimental import pallas as pl
from jax.experimental.pallas import tpu as pltpu
from jax.experimental.pallas import tpu_sc as plsc

D_MODEL = 64
CHUNK = 128


@functools.lru_cache(maxsize=None)
def _make_gather(n_idx: int, d: int, nc: int, ns: int):
    nw = nc * ns
    assert n_idx % (nw * CHUNK) == 0
    per_w = n_idx // nw
    n_chunks = per_w // CHUNK
    mesh = plsc.VectorSubcoreMesh(core_axis_name="c", subcore_axis_name="s")

    @functools.partial(
        pl.kernel,
        mesh=mesh,
        out_type=jax.ShapeDtypeStruct((n_idx, d), jnp.float32),
        scratch_types=[
            pltpu.VMEM((CHUNK,), jnp.int32),
            pltpu.VMEM((CHUNK, d), jnp.float32),
            pltpu.SemaphoreType.DMA,
        ],
        compiler_params=pltpu.CompilerParams(use_tc_tiling_on_sc=False),
    )
    def gather_kernel(idx_hbm, table_hbm, out_hbm, idx_v, rows_v, sem):
        wid = lax.axis_index("s") * nc + lax.axis_index("c")
        base = wid * per_w

        @pl.loop(0, n_chunks)
        def _(i):
            off = base + i * CHUNK
            pltpu.sync_copy(idx_hbm.at[pl.ds(off, CHUNK)], idx_v)
            pltpu.async_copy(table_hbm.at[idx_v], rows_v, sem).wait()
            pltpu.sync_copy(rows_v, out_hbm.at[pl.ds(off, CHUNK)])

    return gather_kernel


@jax.jit
def kernel(x, table):
    idx = x.reshape(-1).astype(jnp.int32)
    info = plsc.get_sparse_core_info()
    gather = _make_gather(idx.shape[0], table.shape[1], info.num_cores, info.num_subcores)
    out = gather(idx, table)
    return out.reshape(x.shape + (table.shape[1],))


# speedup vs baseline: 1.1423x; 1.1423x over previous
import functools

import jax
import jax.numpy as jnp
from jax import lax
from jax.experimental import pallas as pl
from jax.experimental.pallas import tpu as pltpu
from jax.experimental.pallas import tpu_sc as plsc

D_MODEL = 64
CHUNK = 512


@functools.lru_cache(maxsize=None)
def _make_gather(n_idx: int, d: int, nc: int, ns: int):
    nw = nc * ns
    assert n_idx % (nw * CHUNK) == 0
    per_w = n_idx // nw
    n_chunks = per_w // CHUNK
    mesh = plsc.VectorSubcoreMesh(core_axis_name="c", subcore_axis_name="s")

    @functools.partial(
        pl.kernel,
        mesh=mesh,
        out_type=jax.ShapeDtypeStruct((n_idx, d), jnp.float32),
        scratch_types=[
            pltpu.VMEM((CHUNK,), jnp.int32),
            pltpu.VMEM((CHUNK, d), jnp.float32),
            pltpu.SemaphoreType.DMA,
        ],
        compiler_params=pltpu.CompilerParams(use_tc_tiling_on_sc=False),
    )
    def gather_kernel(idx_hbm, table_hbm, out_hbm, idx_v, rows_v, sem):
        wid = lax.axis_index("s") * nc + lax.axis_index("c")
        base = wid * per_w

        @pl.loop(0, n_chunks)
        def _(i):
            off = base + i * CHUNK
            pltpu.sync_copy(idx_hbm.at[pl.ds(off, CHUNK)], idx_v)
            pltpu.async_copy(table_hbm.at[idx_v], rows_v, sem).wait()
            pltpu.sync_copy(rows_v, out_hbm.at[pl.ds(off, CHUNK)])

    return gather_kernel


@jax.jit
def kernel(x, table):
    idx = x.reshape(-1).astype(jnp.int32)
    info = plsc.get_sparse_core_info()
    gather = _make_gather(idx.shape[0], table.shape[1], info.num_cores, info.num_subcores)
    out = gather(idx, table)
    return out.reshape(x.shape + (table.shape[1],))


# speedup vs baseline: 1.1917x; 1.0432x over previous
import functools

import jax
import jax.numpy as jnp
from jax import lax
from jax.experimental import pallas as pl
from jax.experimental.pallas import tpu as pltpu
from jax.experimental.pallas import tpu_sc as plsc

CHUNK = 400
NBUF = 4


@functools.lru_cache(maxsize=None)
def _make_gather(n_idx: int, d: int, nc: int, ns: int):
    nw = nc * ns
    per_w = n_idx // nw
    n_slots = per_w // CHUNK
    n_groups = n_slots // NBUF
    assert per_w % CHUNK == 0 and n_slots % NBUF == 0 and n_groups >= 2
    mesh = plsc.VectorSubcoreMesh(core_axis_name="c", subcore_axis_name="s")

    @functools.partial(
        pl.kernel,
        mesh=mesh,
        out_type=jax.ShapeDtypeStruct((n_idx, d), jnp.float32),
        scratch_types=[
            pltpu.VMEM((per_w,), jnp.int32),
            [pltpu.VMEM((CHUNK, d), jnp.float32) for _ in range(NBUF)],
            [pltpu.SemaphoreType.DMA for _ in range(NBUF)],
            [pltpu.SemaphoreType.DMA for _ in range(NBUF)],
            pltpu.SemaphoreType.DMA,
        ],
        compiler_params=pltpu.CompilerParams(use_tc_tiling_on_sc=False),
    )
    def gather_kernel(idx_hbm, table_hbm, out_hbm, idx_v, rows_v, gat_sem, out_sem, idx_sem):
        wid = lax.axis_index("s") * nc + lax.axis_index("c")
        base = wid * per_w
        pltpu.async_copy(idx_hbm.at[pl.ds(base, per_w)], idx_v, idx_sem).wait()

        def start_gather(t, b):
            pltpu.async_copy(
                table_hbm.at[idx_v.at[pl.ds(t * CHUNK, CHUNK)]],
                rows_v[b],
                gat_sem[b],
            )

        def wait_gather(b):
            pltpu.make_async_copy(
                table_hbm.at[idx_v.at[pl.ds(0, CHUNK)]], rows_v[b], gat_sem[b]
            ).wait()

        def start_out(t, b):
            pltpu.async_copy(
                rows_v[b], out_hbm.at[pl.ds(base + t * CHUNK, CHUNK)], out_sem[b]
            )

        def wait_out(b):
            pltpu.make_async_copy(
                rows_v[b], out_hbm.at[pl.ds(base, CHUNK)], out_sem[b]
            ).wait()

        for b in range(NBUF):
            start_gather(b, b)
            if b >= 2:
                b2 = b - 2
                wait_gather(b2)
                start_out(b - 2, b2)

        @pl.loop(1, n_groups)
        def _(g):
            t0 = g * NBUF
            for b in range(NBUF):
                t = t0 + b
                wait_out(b)
                start_gather(t, b)
                b2 = (b + NBUF - 2) % NBUF
                wait_gather(b2)
                start_out(t - 2, b2)

        last = n_slots - 2
        for k in range(2):
            b2 = (last + k) % NBUF
            wait_gather(b2)
            start_out(last + k, b2)
        for b in range(NBUF):
            wait_out(b)

    return gather_kernel


@jax.jit
def kernel(x, table):
    idx = x.reshape(-1).astype(jnp.int32)
    info = plsc.get_sparse_core_info()
    gather = _make_gather(idx.shape[0], table.shape[1], info.num_cores, info.num_subcores)
    out = gather(idx, table)
    return out.reshape(x.shape + (table.shape[1],))


# speedup vs baseline: 1.2644x; 1.0610x over previous
import functools

import jax
import jax.numpy as jnp
from jax import lax
from jax.experimental import pallas as pl
from jax.experimental.pallas import tpu as pltpu
from jax.experimental.pallas import tpu_sc as plsc

CHUNK = 128
NBUF = 3


@functools.lru_cache(maxsize=None)
def _make(n_idx, d, nc, ns):
    nw = nc * ns
    per_w = n_idx // nw
    T = per_w // CHUNK
    assert per_w % CHUNK == 0 and (T - 2) % NBUF == 0 and T >= 8
    n_groups = (T - 2) // NBUF
    mesh = plsc.VectorSubcoreMesh(core_axis_name="c", subcore_axis_name="s")

    @functools.partial(
        pl.kernel,
        mesh=mesh,
        out_type=jax.ShapeDtypeStruct((n_idx, d), jnp.float32),
        scratch_types=[
            [pltpu.VMEM((CHUNK,), jnp.int32) for _ in range(NBUF)],
            [pltpu.VMEM((CHUNK,), jnp.int32) for _ in range(NBUF)],
            [pltpu.VMEM((CHUNK, 128), jnp.float32) for _ in range(NBUF)],
            [pltpu.VMEM((CHUNK, d), jnp.float32) for _ in range(NBUF)],
            [pltpu.SemaphoreType.DMA for _ in range(NBUF)],
            [pltpu.SemaphoreType.DMA for _ in range(NBUF)],
            [pltpu.SemaphoreType.DMA for _ in range(NBUF)],
        ],
        compiler_params=pltpu.CompilerParams(use_tc_tiling_on_sc=True),
    )
    def k(idx_hbm, pairs_hbm, out_hbm, idx_c, pidx, rows128, rows64,
          ia, ga, oa):
        wid = lax.axis_index("s") * nc + lax.axis_index("c")
        base = wid * per_w

        def start_idx(t, b):
            pltpu.async_copy(idx_hbm.at[pl.ds(base + t * CHUNK, CHUNK)], idx_c[b], ia[b])

        def wait_idx(b):
            pltpu.make_async_copy(idx_hbm.at[pl.ds(base, CHUNK)], idx_c[b], ia[b]).wait()

        def prep(t, b):
            del t
            for v in range(CHUNK // 16):
                s = pl.ds(v * 16, 16)
                pidx[b][s] = lax.shift_right_logical(idx_c[b][s], 1)

        def start_gather(b):
            pltpu.async_copy(pairs_hbm.at[pidx[b]], rows128[b], ga[b])

        def wait_gather(b):
            pltpu.make_async_copy(pairs_hbm.at[pidx[b]], rows128[b], ga[b]).wait()

        def select(b):
            @pl.loop(0, CHUNK // 16)
            def _(g):
                i0 = g * 16
                v = idx_c[b][pl.ds(i0, 16)]
                for l in range(16):
                    off = (v[l] & 1) * 64
                    i = i0 + l
                    for j in range(d // 16):
                        rows64[b][i, pl.ds(j * 16, 16)] = rows128[b][i, pl.ds(off + j * 16, 16)]

        def start_out(t, b):
            pltpu.async_copy(rows64[b], out_hbm.at[pl.ds(base + t * CHUNK, CHUNK)], oa[b])

        def wait_out(b):
            pltpu.make_async_copy(rows64[b], out_hbm.at[pl.ds(base, CHUNK)], oa[b]).wait()

        def complete_prev(t, b1, prefetch):
            wait_gather(b1)
            select(b1)
            start_out(t - 1, b1)
            if prefetch:
                start_idx(t + 2, b1)

        start_idx(0, 0)
        start_idx(1, 1)
        wait_idx(0)
        prep(0, 0)
        start_gather(0)
        start_idx(2, 2)
        wait_idx(1)
        prep(1, 1)
        start_gather(1)
        complete_prev(1, 0, True)

        @pl.loop(0, n_groups)
        def _(g):
            t0 = g * NBUF + 2
            for u in range(NBUF):
                t = t0 + u
                b = (2 + u) % NBUF
                b1 = (b + NBUF - 1) % NBUF
                wait_idx(b)
                prep(t, b)
                if u == 0:
                    @pl.when(t >= NBUF)
                    def _():
                        wait_out(b)
                else:
                    wait_out(b)
                start_gather(b)

                @pl.when(t + 2 <= T - 1)
                def _():
                    complete_prev(t, b1, True)

                @pl.when(t + 2 > T - 1)
                def _():
                    complete_prev(t, b1, False)

        bl = (T - 1) % NBUF
        wait_gather(bl)
        select(bl)
        start_out(T - 1, bl)
        for b in range(NBUF):
            wait_out(b)

    return k


@jax.jit
def kernel(x, table):
    idx = x.reshape(-1).astype(jnp.int32)
    pairs = table.reshape(table.shape[0] // 2, 2 * table.shape[1])
    info = plsc.get_sparse_core_info()
    out = _make(idx.shape[0], table.shape[1], info.num_cores, info.num_subcores)(idx, pairs)
    return out.reshape(x.shape + (table.shape[1],))
